# baseline (speedup 1.0000x reference)
import functools

import jax
import jax.numpy as jnp
from jax import lax
from jax.experimental import pallas as pl
from jax.experimental.pallas import tpu as pltpu
from jax.experimental.pallas import tpu_sc as plsc

N = 10000
E = 320000
D_IN = 128
NHID = 128
D_OUT = 64

NC = 2
NS = 16
C = 80
NCHP = E // NS // C
HALF = 5120
N_PAD = NC * HALF
A_ROWS = HALF + 8
RPT = HALF // NS
IVEC = C // 16
QMAX = NCHP + 2


@functools.cache
def _mesh():
    return plsc.VectorSubcoreMesh(core_axis_name="c", subcore_axis_name="s")


def _zero_fill(buf, width):
    rows = buf.shape[0]
    vecs = width // 16

    def body(k, _):
        i = k // vecs
        j = k % vecs
        buf[i, pl.ds(j * 16, 16)] = jnp.zeros((16,), jnp.float32)
        return 0

    lax.fori_loop(0, rows * vecs, body, 0)


@functools.cache
def _make_hop():
    @functools.partial(
        pl.kernel,
        mesh=_mesh(),
        out_type=jax.ShapeDtypeStruct((N_PAD, NHID), jnp.float32),
        scratch_types=[
            pltpu.VMEM((C,), jnp.int32),
            pltpu.VMEM((C,), jnp.int32),
            pltpu.VMEM((C,), jnp.int32),
            pltpu.VMEM((C,), jnp.int32),
            pltpu.VMEM((C, NHID), jnp.float32),
            pltpu.VMEM((C, NHID), jnp.float32),
            pltpu.VMEM_SHARED((A_ROWS, NHID), jnp.float32),
            pltpu.SemaphoreType.DMA,
            pltpu.SemaphoreType.DMA,
            pltpu.SemaphoreType.DMA,
            pltpu.SemaphoreType.DMA,
        ],
    )
    def hop(u_hbm, rows_hbm, lcol_hbm, y_hbm, ridxa, cidxa, ridxb, cidxb,
            gbufa, gbufb, acc, sema, semb, semia, semib):
        cid = lax.axis_index("c")
        sid = lax.axis_index("s")
        base = cid * HALF

        _zero_fill(gbufa, NHID)
        for b in range(RPT // C):
            pltpu.sync_copy(gbufa, acc.at[pl.ds(sid * RPT + b * C, C)])

        @pl.when(sid == 0)
        def _():
            pltpu.sync_copy(gbufa.at[pl.ds(0, 8)], acc.at[pl.ds(HALF, 8)])

        plsc.subcore_barrier()

        def idx_issue(j, ridx, cidx, semi):
            pltpu.async_copy(rows_hbm.at[sid, j], ridx, semi)
            pltpu.async_copy(lcol_hbm.at[cid, sid, j], cidx, semi)

        def idx_wait(j, ridx, cidx, semi):
            pltpu.make_async_copy(rows_hbm.at[sid, j], ridx, semi).wait()
            pltpu.make_async_copy(lcol_hbm.at[cid, sid, j], cidx, semi).wait()

        pltpu.sync_copy(rows_hbm.at[sid, 0], ridxa)
        pltpu.sync_copy(lcol_hbm.at[cid, sid, 0], cidxa)
        pltpu.async_copy(u_hbm.at[ridxa], gbufa, sema)
        idx_issue(1, ridxb, cidxb, semib)

        def pipe(jj, _):
            j1 = 2 * jj + 1
            j2 = 2 * jj + 2
            j3 = 2 * jj + 3
            idx_wait(j1, ridxb, cidxb, semib)
            pltpu.async_copy(u_hbm.at[ridxb], gbufb, semb)
            pltpu.make_async_copy(u_hbm.at[ridxa], gbufa, sema).wait()
            pltpu.sync_copy(gbufa, acc.at[cidxa], add=True)

            @pl.when(j2 < NCHP)
            def _():
                idx_issue(j2, ridxa, cidxa, semia)
                idx_wait(j2, ridxa, cidxa, semia)
                pltpu.async_copy(u_hbm.at[ridxa], gbufa, sema)

            pltpu.make_async_copy(u_hbm.at[ridxb], gbufb, semb).wait()
            pltpu.sync_copy(gbufb, acc.at[cidxb], add=True)

            @pl.when(j3 < NCHP)
            def _():
                idx_issue(j3, ridxb, cidxb, semib)

            return 0

        lax.fori_loop(0, NCHP // 2, pipe, 0)
        plsc.subcore_barrier()

        for b in range(RPT // C):
            pltpu.sync_copy(acc.at[pl.ds(sid * RPT + b * C, C)], gbufa)
            pltpu.sync_copy(gbufa,
                            y_hbm.at[pl.ds(base + sid * RPT + b * C, C)])

    return hop


@functools.cache
def _make_deg():
    @functools.partial(
        pl.kernel,
        mesh=_mesh(),
        out_type=jax.ShapeDtypeStruct((NC, N_PAD, 16), jnp.float32),
        scratch_types=[
            pltpu.VMEM((C,), jnp.int32),
            pltpu.VMEM((C, 16), jnp.float32),
            pltpu.VMEM_SHARED((N_PAD, 16), jnp.float32),
        ],
    )
    def deg_counts(cols_hbm, out_hbm, cidx, ones, acc):
        cid = lax.axis_index("c")
        sid = lax.axis_index("s")
        rpt = N_PAD // NS

        _zero_fill(ones, 16)
        for b in range(rpt // C):
            pltpu.sync_copy(ones, acc.at[pl.ds(sid * rpt + b * C, C)])

        def fill(i, _):
            ones[i, :] = jnp.ones((16,), jnp.float32)
            return 0

        lax.fori_loop(0, C, fill, 0)
        plsc.subcore_barrier()

        def chunk(j, _):
            pltpu.sync_copy(cols_hbm.at[sid, j], cidx)
            pltpu.sync_copy(ones, acc.at[cidx], add=True)
            return 0

        lax.fori_loop(0, NCHP, chunk, 0)
        plsc.subcore_barrier()

        for b in range(rpt // C):
            pltpu.sync_copy(acc.at[pl.ds(sid * rpt + b * C, C)],
                            out_hbm.at[cid, pl.ds(sid * rpt + b * C, C)])

    return deg_counts


_ER = E // 128
_EB = 2500


def _locidx_body(c_ref, o0_ref, o1_ref):
    c = c_ref[...]
    t = HALF + (c & 7)
    o0_ref[...] = jnp.where(c < HALF, c, t)
    o1_ref[...] = jnp.where(c >= HALF, c - HALF, t)


def _locidx(cols2d):
    return pl.pallas_call(
        _locidx_body,
        grid=(_ER // _EB,),
        in_specs=[pl.BlockSpec((_EB, 128), lambda i: (i, 0))],
        out_specs=[pl.BlockSpec((_EB, 128), lambda i: (i, 0)),
                   pl.BlockSpec((_EB, 128), lambda i: (i, 0))],
        out_shape=[jax.ShapeDtypeStruct((_ER, 128), jnp.int32),
                   jax.ShapeDtypeStruct((_ER, 128), jnp.int32)],
    )(cols2d)


_RB = 1000
_GRID = N // _RB


def _row_spec(d):
    return pl.BlockSpec((_RB, d), lambda i: (i, 0))


def _full_spec(shape):
    return pl.BlockSpec(shape, lambda i: (0,) * len(shape))


def _prep_body(x_ref, w_ref, c0_ref, u_ref, dis_ref, dis2_ref):
    deg = c0_ref[...] + 1.0
    dis = lax.rsqrt(deg)
    z = jnp.dot(x_ref[...], w_ref[...], preferred_element_type=jnp.float32)
    u_ref[...] = dis * z
    dis_ref[...] = dis
    dis2_ref[...] = dis * dis


def _prep(x, W1, c0):
    return pl.pallas_call(
        _prep_body,
        grid=(_GRID,),
        in_specs=[_row_spec(D_IN), _full_spec((D_IN, NHID)), _row_spec(1)],
        out_specs=[_row_spec(NHID), _row_spec(1), _row_spec(1)],
        out_shape=[jax.ShapeDtypeStruct((N, NHID), jnp.float32),
                   jax.ShapeDtypeStruct((N, 1), jnp.float32),
                   jax.ShapeDtypeStruct((N, 1), jnp.float32)],
    )(x, W1, c0)


def _combine_body(y_ref, u_ref, dis_ref, dis2_ref, b_ref, w_ref, f_ref,
                  o_ref):
    t = y_ref[...] + u_ref[...]
    s1 = jnp.where(f_ref[0, 0] > 0.0, dis2_ref[...] * t,
                   dis_ref[...] * t + b_ref[...])
    h = jnp.where(f_ref[0, 1] > 0.0, jax.nn.relu(s1), s1)
    z = dis_ref[...] * jnp.dot(h, w_ref[...],
                               preferred_element_type=jnp.float32)
    o_ref[...] = jnp.where(f_ref[0, 2] > 0.0, z, h)


def _combine(y, u, dis, dis2, b, W, f):
    return pl.pallas_call(
        _combine_body,
        grid=(_GRID,),
        in_specs=[_row_spec(NHID), _row_spec(NHID), _row_spec(1),
                  _row_spec(1), _full_spec((1, NHID)),
                  _full_spec((NHID, NHID)), _full_spec((1, 4))],
        out_specs=_row_spec(NHID),
        out_shape=jax.ShapeDtypeStruct((N, NHID), jnp.float32),
    )(y[:N], u, dis, dis2, b.reshape(1, NHID), W, f.reshape(1, 4))


def _final_body(t_ref, o_ref):
    t = t_ref[...]
    m = jnp.max(t, axis=1, keepdims=True)
    s = jnp.log(jnp.sum(jnp.exp(t - m), axis=1, keepdims=True))
    o_ref[...] = t - m - s


def _final(t):
    return pl.pallas_call(
        _final_body,
        grid=(_GRID,),
        in_specs=[_row_spec(D_OUT)],
        out_specs=_row_spec(D_OUT),
        out_shape=jax.ShapeDtypeStruct((N, D_OUT), jnp.float32),
    )(t)


def kernel(x, edge_index, W1, b1, Wm, bm, W2, b2):
    ei = edge_index.astype(jnp.int32)
    rows = ei[0].reshape(NS, NCHP, C)
    cols = ei[1].reshape(NS, NCHP, C)
    l0, l1 = _locidx(ei[1].reshape(_ER, 128))
    lcol = jnp.stack([l0.reshape(NS, NCHP, C), l1.reshape(NS, NCHP, C)])

    hop = _make_hop()

    counts = _make_deg()(cols)
    c0 = counts[0, :N, :1]

    u0, dis, dis2 = _prep(x, W1, c0)

    zW = jnp.zeros((NHID, NHID), jnp.float32)
    W2p = zW.at[:, :D_OUT].set(W2)
    b2p = jnp.zeros((NHID,), jnp.float32).at[:D_OUT].set(b2)
    zb = jnp.zeros((NHID,), jnp.float32)
    Ws = jnp.stack([zW, Wm, zW, W2p, zW, zW])
    bs = jnp.stack([zb, b1, zb, bm, zb, b2p])
    fs = jnp.array([[1.0, 0.0, 0.0, 0.0],
                    [0.0, 1.0, 1.0, 0.0],
                    [1.0, 0.0, 0.0, 0.0],
                    [0.0, 1.0, 1.0, 0.0],
                    [1.0, 0.0, 0.0, 0.0],
                    [0.0, 0.0, 0.0, 0.0]], jnp.float32)

    def stage(u, wbf):
        W, b, f = wbf
        y = hop(u, rows, lcol)
        return _combine(y, u, dis, dis2, b, W, f), None

    t, _ = lax.scan(stage, u0, (Ws, bs, fs))
    return _final(t[:, :D_OUT])


# speedup vs baseline: 1.2268x; 1.2268x over previous
import functools

import jax
import jax.numpy as jnp
from jax import lax
from jax.experimental import pallas as pl
from jax.experimental.pallas import tpu as pltpu
from jax.experimental.pallas import tpu_sc as plsc

N = 10000
E = 320000
D_IN = 128
NHID = 128
D_OUT = 64

NC = 2
NS = 16
C = 80
NCHP = E // NS // C
HALF = 5120
N_PAD = NC * HALF
A_ROWS = HALF + 8
RPT = HALF // NS
IVEC = C // 16
QMAX = NCHP + 2


@functools.cache
def _mesh():
    return plsc.VectorSubcoreMesh(core_axis_name="c", subcore_axis_name="s")


def _zero_fill(buf, width):
    rows = buf.shape[0]
    vecs = width // 16

    def body(k, _):
        i = k // vecs
        j = k % vecs
        buf[i, pl.ds(j * 16, 16)] = jnp.zeros((16,), jnp.float32)
        return 0

    lax.fori_loop(0, rows * vecs, body, 0)


@functools.cache
def _make_hop():
    @functools.partial(
        pl.kernel,
        mesh=_mesh(),
        out_type=jax.ShapeDtypeStruct((N_PAD, NHID), jnp.float32),
        scratch_types=[
            pltpu.VMEM((C,), jnp.int32),
            pltpu.VMEM((C,), jnp.int32),
            pltpu.VMEM((C,), jnp.int32),
            pltpu.VMEM((C,), jnp.int32),
            pltpu.VMEM((C,), jnp.int32),
            pltpu.VMEM((C,), jnp.int32),
            pltpu.VMEM((C, NHID), jnp.float32),
            pltpu.VMEM((C, NHID), jnp.float32),
            pltpu.VMEM_SHARED((A_ROWS, NHID), jnp.float32),
            pltpu.SemaphoreType.DMA,
            pltpu.SemaphoreType.DMA,
            pltpu.SemaphoreType.DMA,
            pltpu.SemaphoreType.DMA,
            pltpu.SemaphoreType.DMA,
        ],
    )
    def hop(u_hbm, rows_hbm, lcol_hbm, y_hbm, ridxa, cidxa, ridxb, cidxb,
            ridxc, cidxc, gbufa, gbufb, acc, sema, semb, semia, semib,
            semic):
        cid = lax.axis_index("c")
        sid = lax.axis_index("s")
        base = cid * HALF

        _zero_fill(gbufa, NHID)
        for b in range(RPT // C):
            pltpu.sync_copy(gbufa, acc.at[pl.ds(sid * RPT + b * C, C)])

        @pl.when(sid == 0)
        def _():
            pltpu.sync_copy(gbufa.at[pl.ds(0, 8)], acc.at[pl.ds(HALF, 8)])

        plsc.subcore_barrier()

        ridx = [ridxa, ridxb, ridxc]
        cidx = [cidxa, cidxb, cidxc]
        semi = [semia, semib, semic]
        gbuf = [gbufa, gbufb]
        semg = [sema, semb]

        def issue_idx(j, s):
            pltpu.async_copy(rows_hbm.at[sid, j], ridx[s], semi[s])
            pltpu.async_copy(lcol_hbm.at[cid, sid, j], cidx[s], semi[s])

        def wait_idx(j, s):
            pltpu.make_async_copy(rows_hbm.at[sid, j], ridx[s],
                                  semi[s]).wait()
            pltpu.make_async_copy(lcol_hbm.at[cid, sid, j], cidx[s],
                                  semi[s]).wait()

        def start_gather(s, g):
            pltpu.async_copy(u_hbm.at[ridx[s]], gbuf[g], semg[g])

        def wait_gather(s, g):
            pltpu.make_async_copy(u_hbm.at[ridx[s]], gbuf[g], semg[g]).wait()

        def scatter(s, g):
            pltpu.sync_copy(gbuf[g], acc.at[cidx[s]], add=True)

        pltpu.sync_copy(rows_hbm.at[sid, 0], ridxa)
        pltpu.sync_copy(lcol_hbm.at[cid, sid, 0], cidxa)
        start_gather(0, 0)
        issue_idx(1, 1)

        NB6 = (NCHP - 4) // 6

        def pipe(jj, _):
            jb = 6 * jj
            for i in range(6):
                j = jb + i
                issue_idx(j + 2, (i + 2) % 3)
                wait_idx(j + 1, (i + 1) % 3)
                start_gather((i + 1) % 3, (i + 1) % 2)
                wait_gather(i % 3, i % 2)
                scatter(i % 3, i % 2)
            return 0

        lax.fori_loop(0, NB6, pipe, 0)

        j0 = NCHP - 4
        issue_idx(j0 + 2, 2)
        wait_idx(j0 + 1, 1)
        start_gather(1, 1)
        wait_gather(0, 0)
        scatter(0, 0)

        issue_idx(j0 + 3, 0)
        wait_idx(j0 + 2, 2)
        start_gather(2, 0)
        wait_gather(1, 1)
        scatter(1, 1)

        wait_idx(j0 + 3, 0)
        start_gather(0, 1)
        wait_gather(2, 0)
        scatter(2, 0)

        wait_gather(0, 1)
        scatter(0, 1)
        plsc.subcore_barrier()

        for b in range(RPT // C):
            pltpu.sync_copy(acc.at[pl.ds(sid * RPT + b * C, C)], gbufa)
            pltpu.sync_copy(gbufa,
                            y_hbm.at[pl.ds(base + sid * RPT + b * C, C)])

    return hop


@functools.cache
def _make_deg():
    @functools.partial(
        pl.kernel,
        mesh=_mesh(),
        out_type=jax.ShapeDtypeStruct((NC, N_PAD, 16), jnp.float32),
        scratch_types=[
            pltpu.VMEM((C,), jnp.int32),
            pltpu.VMEM((C, 16), jnp.float32),
            pltpu.VMEM_SHARED((N_PAD, 16), jnp.float32),
        ],
    )
    def deg_counts(cols_hbm, out_hbm, cidx, ones, acc):
        cid = lax.axis_index("c")
        sid = lax.axis_index("s")
        rpt = N_PAD // NS

        _zero_fill(ones, 16)
        for b in range(rpt // C):
            pltpu.sync_copy(ones, acc.at[pl.ds(sid * rpt + b * C, C)])

        def fill(i, _):
            ones[i, :] = jnp.ones((16,), jnp.float32)
            return 0

        lax.fori_loop(0, C, fill, 0)
        plsc.subcore_barrier()

        def chunk(j, _):
            pltpu.sync_copy(cols_hbm.at[sid, j], cidx)
            pltpu.sync_copy(ones, acc.at[cidx], add=True)
            return 0

        lax.fori_loop(0, NCHP, chunk, 0)
        plsc.subcore_barrier()

        for b in range(rpt // C):
            pltpu.sync_copy(acc.at[pl.ds(sid * rpt + b * C, C)],
                            out_hbm.at[cid, pl.ds(sid * rpt + b * C, C)])

    return deg_counts


_ER = E // 128
_EB = 2500


def _locidx_body(c_ref, o0_ref, o1_ref):
    c = c_ref[...]
    t = HALF + (c & 7)
    o0_ref[...] = jnp.where(c < HALF, c, t)
    o1_ref[...] = jnp.where(c >= HALF, c - HALF, t)


def _locidx(cols2d):
    return pl.pallas_call(
        _locidx_body,
        grid=(_ER // _EB,),
        in_specs=[pl.BlockSpec((_EB, 128), lambda i: (i, 0))],
        out_specs=[pl.BlockSpec((_EB, 128), lambda i: (i, 0)),
                   pl.BlockSpec((_EB, 128), lambda i: (i, 0))],
        out_shape=[jax.ShapeDtypeStruct((_ER, 128), jnp.int32),
                   jax.ShapeDtypeStruct((_ER, 128), jnp.int32)],
    )(cols2d)


_RB = 1000
_GRID = N // _RB


def _row_spec(d):
    return pl.BlockSpec((_RB, d), lambda i: (i, 0))


def _full_spec(shape):
    return pl.BlockSpec(shape, lambda i: (0,) * len(shape))


def _prep_body(x_ref, w_ref, c0_ref, u_ref, dis_ref, dis2_ref):
    deg = c0_ref[...] + 1.0
    dis = lax.rsqrt(deg)
    z = jnp.dot(x_ref[...], w_ref[...], preferred_element_type=jnp.float32)
    u_ref[...] = dis * z
    dis_ref[...] = dis
    dis2_ref[...] = dis * dis


def _prep(x, W1, c0):
    return pl.pallas_call(
        _prep_body,
        grid=(_GRID,),
        in_specs=[_row_spec(D_IN), _full_spec((D_IN, NHID)), _row_spec(1)],
        out_specs=[_row_spec(NHID), _row_spec(1), _row_spec(1)],
        out_shape=[jax.ShapeDtypeStruct((N, NHID), jnp.float32),
                   jax.ShapeDtypeStruct((N, 1), jnp.float32),
                   jax.ShapeDtypeStruct((N, 1), jnp.float32)],
    )(x, W1, c0)


def _combine_body(y_ref, u_ref, dis_ref, dis2_ref, b_ref, w_ref, f_ref,
                  o_ref):
    t = y_ref[...] + u_ref[...]
    s1 = jnp.where(f_ref[0, 0] > 0.0, dis2_ref[...] * t,
                   dis_ref[...] * t + b_ref[...])
    h = jnp.where(f_ref[0, 1] > 0.0, jax.nn.relu(s1), s1)
    z = dis_ref[...] * jnp.dot(h, w_ref[...],
                               preferred_element_type=jnp.float32)
    o_ref[...] = jnp.where(f_ref[0, 2] > 0.0, z, h)


def _combine(y, u, dis, dis2, b, W, f):
    return pl.pallas_call(
        _combine_body,
        grid=(_GRID,),
        in_specs=[_row_spec(NHID), _row_spec(NHID), _row_spec(1),
                  _row_spec(1), _full_spec((1, NHID)),
                  _full_spec((NHID, NHID)), _full_spec((1, 4))],
        out_specs=_row_spec(NHID),
        out_shape=jax.ShapeDtypeStruct((N, NHID), jnp.float32),
    )(y[:N], u, dis, dis2, b.reshape(1, NHID), W, f.reshape(1, 4))


def _final_body(t_ref, o_ref):
    t = t_ref[...]
    m = jnp.max(t, axis=1, keepdims=True)
    s = jnp.log(jnp.sum(jnp.exp(t - m), axis=1, keepdims=True))
    o_ref[...] = t - m - s


def _final(t):
    return pl.pallas_call(
        _final_body,
        grid=(_GRID,),
        in_specs=[_row_spec(D_OUT)],
        out_specs=_row_spec(D_OUT),
        out_shape=jax.ShapeDtypeStruct((N, D_OUT), jnp.float32),
    )(t)


def kernel(x, edge_index, W1, b1, Wm, bm, W2, b2):
    ei = edge_index.astype(jnp.int32)
    rows = ei[0].reshape(NS, NCHP, C)
    cols = ei[1].reshape(NS, NCHP, C)
    l0, l1 = _locidx(ei[1].reshape(_ER, 128))
    lcol = jnp.stack([l0.reshape(NS, NCHP, C), l1.reshape(NS, NCHP, C)])

    hop = _make_hop()

    counts = _make_deg()(cols)
    c0 = counts[0, :N, :1]

    u0, dis, dis2 = _prep(x, W1, c0)

    zW = jnp.zeros((NHID, NHID), jnp.float32)
    W2p = zW.at[:, :D_OUT].set(W2)
    b2p = jnp.zeros((NHID,), jnp.float32).at[:D_OUT].set(b2)
    zb = jnp.zeros((NHID,), jnp.float32)
    Ws = jnp.stack([zW, Wm, zW, W2p, zW, zW])
    bs = jnp.stack([zb, b1, zb, bm, zb, b2p])
    fs = jnp.array([[1.0, 0.0, 0.0, 0.0],
                    [0.0, 1.0, 1.0, 0.0],
                    [1.0, 0.0, 0.0, 0.0],
                    [0.0, 1.0, 1.0, 0.0],
                    [1.0, 0.0, 0.0, 0.0],
                    [0.0, 0.0, 0.0, 0.0]], jnp.float32)

    def stage(u, wbf):
        W, b, f = wbf
        y = hop(u, rows, lcol)
        return _combine(y, u, dis, dis2, b, W, f), None

    t, _ = lax.scan(stage, u0, (Ws, bs, fs))
    return _final(t[:, :D_OUT])
